# Initial kernel scaffold; baseline (speedup 1.0000x reference)
import jax
import jax.numpy as jnp
from jax.experimental import pallas as pl


def kernel(x, table, scale):
    raise NotImplementedError("write your pallas kernel here")


# speedup vs baseline: 4.6701x; 4.6701x over previous
import functools
import jax
import jax.numpy as jnp
from jax import lax
from jax.experimental import pallas as pl
from jax.experimental.pallas import tpu as pltpu
from jax.experimental.pallas import tpu_sc as plsc

_L = 16


@functools.lru_cache(maxsize=None)
def _build(B, V, D, dtype_name):
    info = plsc.get_sparse_core_info()
    NC, NS = info.num_cores, info.num_subcores
    NW = NC * NS
    C = 128
    assert B % (NW * C) == 0
    G = B // (NW * C)
    b_per_w = B // NW
    dtype = jnp.dtype(dtype_name)

    mesh = plsc.VectorSubcoreMesh(core_axis_name="c", subcore_axis_name="s")

    @functools.partial(
        pl.kernel,
        mesh=mesh,
        out_type=jax.ShapeDtypeStruct((B, D), dtype),
        scratch_types=[
            pltpu.VMEM((G, C), jnp.int32),
            pltpu.VMEM((C, D), dtype),
            pltpu.VMEM((_L,), dtype),
            pltpu.SemaphoreType.DMA,
        ],
    )
    def emb_kernel(idx_hbm, table_hbm, scale_hbm, out_hbm, idx_v, buf, scl_v, gsem):
        wid = lax.axis_index("s") * NC + lax.axis_index("c")
        pltpu.sync_copy(idx_hbm.at[wid], idx_v)
        pltpu.sync_copy(scale_hbm, scl_v)
        s = scl_v[...]

        def chunk_body(g, carry):
            pltpu.async_copy(table_hbm.at[idx_v.at[g]], buf, gsem).wait()

            def row_body(r, carry2):
                for j in range(D // _L):
                    sl = pl.ds(j * _L, _L)
                    buf[r, sl] = buf[r, sl] * s
                return carry2

            lax.fori_loop(0, C, row_body, 0)
            pltpu.sync_copy(buf, out_hbm.at[pl.ds(wid * b_per_w + g * C, C)])
            return carry

        lax.fori_loop(0, G, chunk_body, 0)

    return emb_kernel, NW, C, G


def kernel(x, table, scale):
    Bt, S = x.shape
    V, D = table.shape
    B = Bt * S
    emb_kernel, NW, C, G = _build(B, V, D, table.dtype.name)
    idx2 = x.reshape(NW, G, C).astype(jnp.int32)
    scale_v = jnp.full((_L,), scale, dtype=table.dtype)
    out = emb_kernel(idx2, table, scale_v)
    return out.reshape(Bt, S, D)


# speedup vs baseline: 7.6797x; 1.6444x over previous
import functools
import jax
import jax.numpy as jnp
from jax import lax
from jax.experimental import pallas as pl
from jax.experimental.pallas import tpu as pltpu
from jax.experimental.pallas import tpu_sc as plsc

_L = 16


@functools.lru_cache(maxsize=None)
def _build(B, V, D, dtype_name):
    info = plsc.get_sparse_core_info()
    NC, NS = info.num_cores, info.num_subcores
    NW = NC * NS
    C = 128
    assert B % (NW * C) == 0
    G = B // (NW * C)
    assert G % 2 == 0
    T = G // 2
    b_per_w = B // NW
    dtype = jnp.dtype(dtype_name)

    mesh = plsc.VectorSubcoreMesh(core_axis_name="c", subcore_axis_name="s")

    @functools.partial(
        pl.kernel,
        mesh=mesh,
        out_type=jax.ShapeDtypeStruct((B, D), dtype),
        scratch_types=[
            pltpu.VMEM((G, C), jnp.int32),
            pltpu.VMEM((C, D), dtype),
            pltpu.VMEM((C, D), dtype),
            pltpu.VMEM((C, D), dtype),
            pltpu.VMEM((C, D), dtype),
            pltpu.VMEM((_L,), dtype),
            pltpu.SemaphoreType.DMA,
            pltpu.SemaphoreType.DMA,
            pltpu.SemaphoreType.DMA,
            pltpu.SemaphoreType.DMA,
        ],
    )
    def emb_kernel(idx_hbm, table_hbm, scale_hbm, out_hbm,
                   idx_v, in_a, in_b, out_a, out_b, scl_v,
                   gi_a, gi_b, go_a, go_b):
        wid = lax.axis_index("s") * NC + lax.axis_index("c")
        base = wid * b_per_w
        pltpu.sync_copy(idx_hbm.at[wid], idx_v)
        pltpu.sync_copy(scale_hbm, scl_v)
        s = scl_v[...]

        def gather_start(g, buf, sem):
            pltpu.async_copy(table_hbm.at[idx_v.at[g]], buf, sem)

        def gather_wait(buf, sem):
            pltpu.make_async_copy(table_hbm.at[pl.ds(0, C)], buf, sem).wait()

        def out_start(g, buf, sem):
            pltpu.async_copy(buf, out_hbm.at[pl.ds(base + g * C, C)], sem)

        def out_wait(buf, sem):
            pltpu.make_async_copy(buf, out_hbm.at[pl.ds(0, C)], sem).wait()

        def scale(src, dst):
            def row_body(r, carry):
                for j in range(D // _L):
                    sl = pl.ds(j * _L, _L)
                    dst[r, sl] = src[r, sl] * s
                return carry
            lax.fori_loop(0, C, row_body, 0)

        gather_start(0, in_a, gi_a)
        gather_start(1, in_b, gi_b)

        def body(t, carry):
            def half(g, in_buf, out_buf, gi, go):
                gather_wait(in_buf, gi)

                @pl.when(t > 0)
                def _():
                    out_wait(out_buf, go)

                scale(in_buf, out_buf)
                out_start(g, out_buf, go)

                @pl.when(t + 1 < T)
                def _():
                    gather_start(g + 2, in_buf, gi)

            half(2 * t, in_a, out_a, gi_a, go_a)
            half(2 * t + 1, in_b, out_b, gi_b, go_b)
            return carry

        lax.fori_loop(0, T, body, 0)
        out_wait(out_a, go_a)
        out_wait(out_b, go_b)

    return emb_kernel, NW, C, G


def kernel(x, table, scale):
    Bt, S = x.shape
    V, D = table.shape
    B = Bt * S
    emb_kernel, NW, C, G = _build(B, V, D, table.dtype.name)
    idx3 = x.reshape(NW, G, C).astype(jnp.int32)
    scale_v = jnp.full((_L,), scale, dtype=table.dtype)
    out = emb_kernel(idx3, table, scale_v)
    return out.reshape(Bt, S, D)
